# Initial kernel scaffold; baseline (speedup 1.0000x reference)
import jax
import jax.numpy as jnp
from jax.experimental import pallas as pl


def kernel(graph_feat, batch, edge_index):
    raise NotImplementedError("write your pallas kernel here")


# speedup vs baseline: 5.5833x; 5.5833x over previous
import functools

import jax
import jax.numpy as jnp
from jax import lax
from jax.experimental import pallas as pl
from jax.experimental.pallas import tpu as pltpu
from jax.experimental.pallas import tpu_sc as plsc

NUM_GRAPHS = 256
N_NODES = 10000
N_EDGES = 320000
D_FEAT = 128

NC = 2
NS = 16
L = 16
NW = NC * NS
E_W = N_EDGES // NW
C = 80
NCHUNK = E_W // C

_mesh = plsc.VectorSubcoreMesh(core_axis_name="c", subcore_axis_name="s")


@functools.partial(
    pl.kernel,
    mesh=_mesh,
    out_type=jax.ShapeDtypeStruct((N_EDGES, D_FEAT), jnp.float32),
    scratch_types=[
        pltpu.VMEM((C,), jnp.int32),
        pltpu.VMEM((C,), jnp.int32),
        pltpu.VMEM((C, D_FEAT), jnp.float32),
        pltpu.SemaphoreType.DMA,
    ],
)
def _unpool(gf_hbm, batch_hbm, esrc_hbm, out_hbm,
            idx_v, eb_v, rows_v, gsem):
    wid = lax.axis_index("s") * NC + lax.axis_index("c")
    base = wid * E_W

    def chunk_body(i, carry):
        off = base + i * C
        pltpu.sync_copy(esrc_hbm.at[pl.ds(off, C)], idx_v)
        pltpu.async_copy(batch_hbm.at[idx_v], eb_v, gsem).wait()
        pltpu.async_copy(gf_hbm.at[eb_v], rows_v, gsem).wait()
        pltpu.sync_copy(rows_v, out_hbm.at[pl.ds(off, C)])
        return carry

    lax.fori_loop(0, NCHUNK, chunk_body, 0)


def kernel(graph_feat, batch, edge_index):
    edge_src = edge_index[0]
    return _unpool(graph_feat, batch, edge_src)


# speedup vs baseline: 6.3756x; 1.1419x over previous
import functools

import jax
import jax.numpy as jnp
from jax import lax
from jax.experimental import pallas as pl
from jax.experimental.pallas import tpu as pltpu
from jax.experimental.pallas import tpu_sc as plsc

NUM_GRAPHS = 256
N_NODES = 10000
N_EDGES = 320000
D_FEAT = 128

NC = 2
NS = 16
NW = NC * NS
E_W = N_EDGES // NW
C = 80
KB = 5
NSUP = E_W // (C * KB)

_mesh = plsc.VectorSubcoreMesh(core_axis_name="c", subcore_axis_name="s")


@functools.partial(
    pl.kernel,
    mesh=_mesh,
    out_type=jax.ShapeDtypeStruct((N_EDGES, D_FEAT), jnp.float32),
    scratch_types=[
        pltpu.VMEM((E_W,), jnp.int32),
        pltpu.VMEM((E_W,), jnp.int32),
        pltpu.VMEM((KB, C, D_FEAT), jnp.float32),
        pltpu.VMEM((KB, C, D_FEAT), jnp.float32),
        pltpu.SemaphoreType.DMA,
        pltpu.SemaphoreType.DMA,
        pltpu.SemaphoreType.DMA,
    ],
)
def _unpool(gf_hbm, batch_hbm, esrc_hbm, out_hbm,
            idx_full, eb_full, rows0, rows1, sem_a, sem_g, sem_o):
    wid = lax.axis_index("s") * NC + lax.axis_index("c")
    base = wid * E_W

    pltpu.sync_copy(esrc_hbm.at[pl.ds(base, E_W)], idx_full)

    def fire_a(g, carry):
        for j in range(KB):
            lo = (g * KB + j) * C
            pltpu.async_copy(batch_hbm.at[idx_full.at[pl.ds(lo, C)]],
                             eb_full.at[pl.ds(lo, C)], sem_a)
        return carry

    lax.fori_loop(0, NSUP, fire_a, 0)

    def drain_a(g, carry):
        for j in range(KB):
            lo = (g * KB + j) * C
            pltpu.make_async_copy(batch_hbm.at[idx_full.at[pl.ds(lo, C)]],
                                  eb_full.at[pl.ds(lo, C)], sem_a).wait()
        return carry

    lax.fori_loop(0, NSUP, drain_a, 0)

    def fire_g(s, buf):
        for j in range(KB):
            lo = (s * KB + j) * C
            pltpu.async_copy(gf_hbm.at[eb_full.at[pl.ds(lo, C)]],
                             buf.at[j], sem_g)

    def drain_g(s, buf):
        for j in range(KB):
            lo = (s * KB + j) * C
            pltpu.make_async_copy(gf_hbm.at[eb_full.at[pl.ds(lo, C)]],
                                  buf.at[j], sem_g).wait()

    def fire_o(s, buf):
        for j in range(KB):
            off = base + (s * KB + j) * C
            pltpu.async_copy(buf.at[j], out_hbm.at[pl.ds(off, C)], sem_o)

    def drain_o(s, buf):
        for j in range(KB):
            off = base + (s * KB + j) * C
            pltpu.make_async_copy(buf.at[j], out_hbm.at[pl.ds(off, C)],
                                  sem_o).wait()

    fire_g(0, rows0)
    drain_g(0, rows0)
    fire_o(0, rows0)
    fire_g(1, rows1)

    def body(t, carry):
        s = 2 * t + 2
        drain_g(s - 1, rows1)
        fire_o(s - 1, rows1)
        drain_o(s - 2, rows0)
        fire_g(s, rows0)
        drain_g(s, rows0)
        fire_o(s, rows0)
        drain_o(s - 1, rows1)
        fire_g(s + 1, rows1)
        return carry

    lax.fori_loop(0, (NSUP - 3) // 2, body, 0)

    s = NSUP - 1
    drain_g(s - 1, rows1)
    fire_o(s - 1, rows1)
    drain_o(s - 2, rows0)
    fire_g(s, rows0)
    drain_g(s, rows0)
    fire_o(s, rows0)
    drain_o(s - 1, rows1)
    drain_o(s, rows0)


def kernel(graph_feat, batch, edge_index):
    edge_src = edge_index[0]
    return _unpool(graph_feat, batch, edge_src)


# speedup vs baseline: 22.7281x; 3.5648x over previous
import functools

import jax
import jax.numpy as jnp
from jax import lax
from jax.experimental import pallas as pl
from jax.experimental.pallas import tpu as pltpu
from jax.experimental.pallas import tpu_sc as plsc

NUM_GRAPHS = 256
N_NODES = 10000
N_EDGES = 320000
D_FEAT = 128

NC = 2
NS = 16
NW = NC * NS
E_W = N_EDGES // NW
C = 80
NR = 5
NG = E_W // (C * NR)

_mesh = plsc.VectorSubcoreMesh(core_axis_name="c", subcore_axis_name="s")


@functools.partial(
    pl.kernel,
    mesh=_mesh,
    out_type=jax.ShapeDtypeStruct((N_EDGES, D_FEAT), jnp.float32),
    scratch_types=[
        pltpu.VMEM_SHARED((NUM_GRAPHS, D_FEAT), jnp.float32),
        pltpu.VMEM_SHARED((N_NODES,), jnp.int32),
        pltpu.VMEM((E_W,), jnp.int32),
        pltpu.VMEM((E_W,), jnp.int32),
        pltpu.VMEM((NR, C, D_FEAT), jnp.float32),
        pltpu.SemaphoreType.DMA,
        pltpu.SemaphoreType.DMA,
        pltpu.SemaphoreType.DMA,
    ],
)
def _unpool(gf_hbm, batch_hbm, esrc_hbm, out_hbm,
            gf_sh, batch_sh, idx_full, eb_full, rows, sem_a, sem_g, sem_o):
    sid = lax.axis_index("s")
    wid = sid * NC + lax.axis_index("c")
    base = wid * E_W

    @pl.when(sid == 0)
    def _stage():
        pltpu.async_copy(gf_hbm, gf_sh, sem_g)
        pltpu.async_copy(batch_hbm, batch_sh, sem_a)
        pltpu.make_async_copy(gf_hbm, gf_sh, sem_g).wait()
        pltpu.make_async_copy(batch_hbm, batch_sh, sem_a).wait()

    pltpu.sync_copy(esrc_hbm.at[pl.ds(base, E_W)], idx_full)
    plsc.subcore_barrier()

    def fire_a(g, carry):
        for j in range(NR):
            lo = (g * NR + j) * C
            pltpu.async_copy(batch_sh.at[idx_full.at[pl.ds(lo, C)]],
                             eb_full.at[pl.ds(lo, C)], sem_a)
        return carry

    lax.fori_loop(0, NG, fire_a, 0)

    def drain_a(g, carry):
        for j in range(NR):
            lo = (g * NR + j) * C
            pltpu.make_async_copy(batch_sh.at[idx_full.at[pl.ds(lo, C)]],
                                  eb_full.at[pl.ds(lo, C)], sem_a).wait()
        return carry

    lax.fori_loop(0, NG, drain_a, 0)

    def gather_rows(c, r):
        lo = c * C
        pltpu.async_copy(gf_sh.at[eb_full.at[pl.ds(lo, C)]],
                         rows.at[r], sem_g)
        pltpu.make_async_copy(gf_sh.at[eb_full.at[pl.ds(lo, C)]],
                              rows.at[r], sem_g).wait()

    def fire_o(c, r):
        off = base + c * C
        pltpu.async_copy(rows.at[r], out_hbm.at[pl.ds(off, C)], sem_o)

    def drain_o(c, r):
        off = base + c * C
        pltpu.make_async_copy(rows.at[r], out_hbm.at[pl.ds(off, C)],
                              sem_o).wait()

    for r in range(NR):
        gather_rows(r, r)
        fire_o(r, r)

    def body(g, carry):
        for r in range(NR):
            c = g * NR + r
            drain_o(c - NR, r)
            gather_rows(c, r)
            fire_o(c, r)
        return carry

    lax.fori_loop(1, NG, body, 0)

    for r in range(NR):
        drain_o((NG - 1) * NR + r, r)


def kernel(graph_feat, batch, edge_index):
    edge_src = edge_index[0]
    return _unpool(graph_feat, batch, edge_src)
